# baseline (speedup 1.0000x reference)
import functools

import jax
import jax.numpy as jnp
from jax import lax
from jax.experimental import pallas as pl
from jax.experimental.pallas import tpu as pltpu
from jax.experimental.pallas import tpu_sc as plsc

MAXV = 1024
NC = 2
NS = 16
NW = NC * NS
NB = 2


def _make_kernel(B, D, P):
    b_per_w = B // NW
    n_chunks = b_per_w // NB
    nrows = NB * 2 * P
    wpr = D // 2
    mesh = plsc.VectorSubcoreMesh(core_axis_name="c", subcore_axis_name="s")

    @functools.partial(
        pl.kernel,
        mesh=mesh,
        out_type=jax.ShapeDtypeStruct((B, P, D), jnp.float32),
        compiler_params=pltpu.CompilerParams(use_tc_tiling_on_sc=False,
                                             needs_layout_passes=False),
        scratch_types=[
            pltpu.VMEM((MAXV, wpr), jnp.int32),
            pltpu.VMEM((2, NB, 3, P), jnp.int32),
            pltpu.VMEM((2, nrows), jnp.int32),
            pltpu.VMEM((2, nrows, wpr), jnp.int32),
            pltpu.VMEM((2, P, D), jnp.float32),
            pltpu.SemaphoreType.DMA,
            pltpu.SemaphoreType.DMA,
            pltpu.SemaphoreType.DMA,
            pltpu.SemaphoreType.DMA,
            pltpu.SemaphoreType.DMA,
            pltpu.SemaphoreType.DMA,
        ],
    )
    def k(idx_hbm, tbl_hbm, out_hbm, tbl0_v, idx_v, list_v, rows_v, out_v,
          si0, si1, sg0, sg1, so0, so1):
        cid = lax.axis_index("c")
        sid = lax.axis_index("s")
        wid = sid * NC + cid
        sis = (si0, si1)
        sgs = (sg0, sg1)
        sos = (so0, so1)

        ji = lax.iota(jnp.int32, 16)
        p2c = [[(ji + 16 * ch) * 2 + (c - 1) for c in (1, 2)]
               for ch in range(P // 16)]
        evens = [ji * 2 + 32 * kk for kk in range(wpr // 16)]

        def idx_copy(g, buf):
            b0 = wid * b_per_w + g * NB
            return pltpu.make_async_copy(
                idx_hbm.at[pl.ds(b0, NB)], idx_v.at[buf], sis[buf])

        def row_gather(buf):
            return pltpu.make_async_copy(
                tbl_hbm.at[list_v.at[buf]], rows_v.at[buf], sgs[buf])

        def out_copy(g, bl):
            b0 = wid * b_per_w + g * NB + bl
            return pltpu.make_async_copy(
                out_v.at[bl], out_hbm.at[b0], sos[bl])

        def build_list(buf):
            for bl in range(NB):
                dst = list_v.at[buf, pl.ds(bl * 2 * P, 2 * P)]
                for ch in range(P // 16):
                    for c in (1, 2):
                        iv = idx_v[buf, bl, c, pl.ds(ch * 16, 16)]
                        iv = iv + (c * MAXV)
                        plsc.store_scatter(dst, [p2c[ch][c - 1]], iv)

        pltpu.sync_copy(tbl_hbm.at[pl.ds(0, MAXV)], tbl0_v)
        idx_copy(0, 0).start()
        idx_copy(1, 1).start()
        idx_copy(0, 0).wait()
        build_list(0)
        row_gather(0).start()

        def pair_body(pp, carry):
          for buf in range(2):
            g = pp * 2 + buf
            nxt = 1 - buf

            @pl.when(g + 1 < n_chunks)
            def _():
                idx_copy(g + 1, nxt).wait()
                build_list(nxt)
                row_gather(nxt).start()

            row_gather(buf).wait()

            for bl in range(NB):
                @pl.when(g >= 1)
                def _():
                    out_copy(g - 1, bl).wait()

                rbase = bl * 2 * P
                iv0s = [idx_v[buf, bl, 0, pl.ds(ch * 16, 16)]
                        for ch in range(P // 16)]

                def row_words(p):
                    s0 = iv0s[p // 16][p % 16]
                    c0 = [tbl0_v[s0, pl.ds(kk * 16, 16)]
                          for kk in range(wpr // 16)]
                    c12 = [[rows_v[buf, rbase + p * 2 + c1, pl.ds(kk * 16, 16)]
                            for kk in range(wpr // 16)] for c1 in range(2)]
                    return [c0] + c12

                def emit_pixel(p, ws):
                    dst = out_v.at[bl, p]
                    for kk in range(wpr // 16):
                        acc = (plsc.bitcast(ws[0][kk], jnp.bfloat16)
                               + plsc.bitcast(ws[1][kk], jnp.bfloat16))
                        acc = acc + plsc.bitcast(ws[2][kk], jnp.bfloat16)
                        lo, hi = plsc.unpack(
                            acc, format=plsc.PackFormat.INTERLEAVED)
                        plsc.store_scatter(dst, [evens[kk]], lo)
                        plsc.store_scatter(dst, [evens[kk] + 1], hi)

                prev = row_words(0)
                for p in range(1, P):
                    cur = row_words(p)
                    emit_pixel(p - 1, prev)
                    prev = cur
                emit_pixel(P - 1, prev)
                out_copy(g, bl).start()

            @pl.when(g + 2 < n_chunks)
            def _():
                idx_copy(g + 2, buf).start()
          return carry

        lax.fori_loop(0, n_chunks // 2, pair_body, 0)
        out_copy(n_chunks - 1, 0).wait()
        out_copy(n_chunks - 1, 1).wait()

    return k


def kernel(inputs, table):
    B, C, H, W = inputs.shape
    V3, D = table.shape
    P = H * W
    tbf = table.astype(jnp.bfloat16)
    tw = jax.lax.bitcast_convert_type(
        tbf.reshape(V3, D // 2, 2), jnp.int32)
    idx = inputs.reshape(B, C, P)
    out = _make_kernel(B, D, P)(idx, tw)
    return jnp.transpose(out.reshape(B, H, W, D), (0, 3, 1, 2))


# speedup vs baseline: 1.6898x; 1.6898x over previous
import functools

import jax
import jax.numpy as jnp
from jax import lax
from jax.experimental import pallas as pl
from jax.experimental.pallas import tpu as pltpu
from jax.experimental.pallas import tpu_sc as plsc

MAXV = 1024
NC = 2
NS = 16
NW = NC * NS
NB = 2


def _make_kernel(B, D, P):
    b_per_w = B // NW
    n_chunks = b_per_w // NB
    nrows = NB * 3 * P
    wpr = D // 2
    mesh = plsc.VectorSubcoreMesh(core_axis_name="c", subcore_axis_name="s")

    @functools.partial(
        pl.kernel,
        mesh=mesh,
        out_type=jax.ShapeDtypeStruct((B, P, D), jnp.float32),
        compiler_params=pltpu.CompilerParams(use_tc_tiling_on_sc=False,
                                             needs_layout_passes=False),
        scratch_types=[
            pltpu.VMEM((2, NB, 3, P), jnp.int32),
            pltpu.VMEM((2, nrows), jnp.int32),
            pltpu.VMEM((2, nrows, wpr), jnp.int32),
            pltpu.VMEM((2, NB, P, D), jnp.float32),
            pltpu.SemaphoreType.DMA,
            pltpu.SemaphoreType.DMA,
            pltpu.SemaphoreType.DMA,
            pltpu.SemaphoreType.DMA,
            pltpu.SemaphoreType.DMA,
            pltpu.SemaphoreType.DMA,
        ],
    )
    def k(idx_hbm, tbl_hbm, out_hbm, idx_v, list_v, rows_v, out_v,
          si0, si1, sg0, sg1, so0, so1):
        cid = lax.axis_index("c")
        sid = lax.axis_index("s")
        wid = sid * NC + cid
        sis = (si0, si1)
        sgs = (sg0, sg1)
        sos = (so0, so1)

        ji = lax.iota(jnp.int32, 16)
        p3c = [[(ji + 16 * ch) * 3 + c for c in range(3)]
               for ch in range(P // 16)]
        evens = [ji * 2 + 32 * kk for kk in range(wpr // 16)]

        def idx_copy(g, buf):
            b0 = wid * b_per_w + g * NB
            return pltpu.make_async_copy(
                idx_hbm.at[pl.ds(b0, NB)], idx_v.at[buf], sis[buf])

        def row_gather(buf):
            return pltpu.make_async_copy(
                tbl_hbm.at[list_v.at[buf]], rows_v.at[buf], sgs[buf])

        def out_copy(g, buf):
            b0 = wid * b_per_w + g * NB
            return pltpu.make_async_copy(
                out_v.at[buf], out_hbm.at[pl.ds(b0, NB)], sos[buf])

        def build_list(buf):
            for bl in range(NB):
                dst = list_v.at[buf, pl.ds(bl * 3 * P, 3 * P)]
                for ch in range(P // 16):
                    for c in range(3):
                        iv = idx_v[buf, bl, c, pl.ds(ch * 16, 16)]
                        if c:
                            iv = iv + (c * MAXV)
                        plsc.store_scatter(dst, [p3c[ch][c]], iv)

        idx_copy(0, 0).start()
        idx_copy(1, 1).start()
        idx_copy(0, 0).wait()
        build_list(0)
        row_gather(0).start()

        def pair_body(pp, carry):
          for buf in range(2):
            g = pp * 2 + buf
            nxt = 1 - buf

            @pl.when(g + 1 < n_chunks)
            def _():
                idx_copy(g + 1, nxt).wait()
                build_list(nxt)
                row_gather(nxt).start()

            @pl.when(g + 2 < n_chunks)
            def _():
                idx_copy(g + 2, buf).start()

            row_gather(buf).wait()

            @pl.when(g >= 2)
            def _():
                out_copy(g - 2, buf).wait()

            def b_body(bl, carry2):
                rbase = bl * 3 * P

                def row_words(p, c):
                    return [rows_v[buf, rbase + p * 3 + c, pl.ds(kk * 16, 16)]
                            for kk in range(wpr // 16)]

                def emit_pixel(p, ws):
                    dst = out_v.at[buf, bl, p]
                    for kk in range(wpr // 16):
                        acc = (plsc.bitcast(ws[0][kk], jnp.bfloat16)
                               + plsc.bitcast(ws[1][kk], jnp.bfloat16))
                        acc = acc + plsc.bitcast(ws[2][kk], jnp.bfloat16)
                        lo, hi = plsc.unpack(
                            acc, format=plsc.PackFormat.INTERLEAVED)
                        plsc.store_scatter(dst, [evens[kk]], lo)
                        plsc.store_scatter(dst, [evens[kk] + 1], hi)

                prev = [row_words(0, c) for c in range(3)]
                for p in range(1, P):
                    cur = [row_words(p, c) for c in range(3)]
                    emit_pixel(p - 1, prev)
                    prev = cur
                emit_pixel(P - 1, prev)
                return carry2

            lax.fori_loop(0, NB, b_body, 0)
            out_copy(g, buf).start()
          return carry

        lax.fori_loop(0, n_chunks // 2, pair_body, 0)
        out_copy(n_chunks - 2, 0).wait()
        out_copy(n_chunks - 1, 1).wait()

    return k


def kernel(inputs, table):
    B, C, H, W = inputs.shape
    V3, D = table.shape
    P = H * W
    tbf = table.astype(jnp.bfloat16)
    tw = jax.lax.bitcast_convert_type(
        tbf.reshape(V3, D // 2, 2), jnp.int32)
    idx = inputs.reshape(B, C, P)
    out = _make_kernel(B, D, P)(idx, tw)
    return jnp.transpose(out.reshape(B, H, W, D), (0, 3, 1, 2))
